# baseline (speedup 1.0000x reference)
import functools

import jax
import jax.numpy as jnp
from jax import lax
from jax.experimental import pallas as pl
from jax.experimental.pallas import tpu as pltpu
from jax.experimental.pallas import tpu_sc as plsc

N = 10000
D = 128
H = 128
C = 16
E = 320000

NC = 2
NS = 16
NW = NC * NS

N_PAD = 10112
ZR = N_PAD // NS
K = 128
NCH = 160
EPT = NCH * K
E_PAD = NS * EPT
DH = D // 2
KB = 4

_mesh = plsc.VectorSubcoreMesh(core_axis_name="c", subcore_axis_name="s")


DW = 8


def _make_agg(with_deg):
    out_type = [jax.ShapeDtypeStruct((NC, N_PAD, DH), jnp.float32)]
    scratch = [
        pltpu.VMEM((KB, K), jnp.int32),
        pltpu.VMEM((NCH, K), jnp.int32),
        pltpu.VMEM((K, DH), jnp.float32),
        pltpu.VMEM((K, DH), jnp.float32),
        pltpu.VMEM((K, DH), jnp.float32),
        pltpu.VMEM((K, DH), jnp.float32),
        pltpu.VMEM_SHARED((N_PAD, DH), jnp.float32),
    ] + [pltpu.SemaphoreType.DMA] * (3 * KB)
    if with_deg:
        out_type.append(jax.ShapeDtypeStruct((NC, N_PAD, DW), jnp.float32))
        scratch.append(pltpu.VMEM((K, DW), jnp.float32))
        scratch.append(pltpu.VMEM_SHARED((N_PAD, DW), jnp.float32))

    def body(h_hbm, srcr, dstr, z2d, zdeg, ones_hbm, *rest):
        if with_deg:
            out_hbm, deg_hbm, sidxr, dall, r0, r1, r2, r3, acc = rest[:9]
            sems = rest[9:9 + 3 * KB]
            onesb, dacc = rest[9 + 3 * KB:]
        else:
            out_hbm, sidxr, dall, r0, r1, r2, r3, acc = rest[:8]
            sems = rest[8:8 + 3 * KB]
        isem = sems[:KB]
        gsem = sems[KB:2 * KB]
        ssem = sems[2 * KB:]
        rows = (r0, r1, r2, r3)
        c = lax.axis_index("c")
        s = lax.axis_index("s")

        pltpu.sync_copy(dstr.at[s], dall)
        pltpu.sync_copy(z2d, acc.at[pl.ds(s * ZR, ZR)])
        if with_deg:
            pltpu.sync_copy(ones_hbm, onesb)
            pltpu.sync_copy(zdeg, dacc.at[pl.ds(s * ZR, ZR)])
        plsc.subcore_barrier()

        tbl = h_hbm.at[c]
        gdon = tbl.at[pl.ds(0, K)]
        idon = srcr.at[s, 0]

        def sdrain(b):
            pltpu.make_async_copy(rows[b], acc.at[dall.at[0]], ssem[b]).wait()

        for b in range(KB):
            pltpu.async_copy(srcr.at[s, b], sidxr.at[b], isem[b])
        for b in range(2):
            pltpu.make_async_copy(idon, sidxr.at[b], isem[b]).wait()
            pltpu.async_copy(tbl.at[sidxr.at[b]], rows[b], gsem[b])

        def it(i, carry):
            for b in range(KB):
                j = i * KB + b
                bg = (b + 2) % KB
                pltpu.make_async_copy(gdon, rows[b], gsem[b]).wait()

                @pl.when(j + KB < NCH)
                def _():
                    pltpu.async_copy(srcr.at[s, j + KB], sidxr.at[b], isem[b])

                pltpu.async_copy(rows[b], acc.at[dall.at[j]], ssem[b], add=True)
                if with_deg:

                    @pl.when(c == b % 2)
                    def _():
                        pltpu.sync_copy(onesb, dacc.at[dall.at[j]], add=True)

                @pl.when(j + 2 < NCH)
                def _():
                    @pl.when(j >= 2)
                    def _():
                        sdrain(bg)

                    pltpu.make_async_copy(idon, sidxr.at[bg], isem[bg]).wait()
                    pltpu.async_copy(tbl.at[sidxr.at[bg]], rows[bg], gsem[bg])

            return carry

        lax.fori_loop(0, NCH // KB, it, 0)
        for b in range(KB):
            sdrain(b)
        plsc.subcore_barrier()
        pltpu.sync_copy(acc.at[pl.ds(s * ZR, ZR)], out_hbm.at[c, pl.ds(s * ZR, ZR)])
        if with_deg:
            pltpu.sync_copy(dacc.at[pl.ds(s * ZR, ZR)], deg_hbm.at[c, pl.ds(s * ZR, ZR)])

    return pl.kernel(
        body,
        out_type=out_type if with_deg else out_type[0],
        mesh=_mesh,
        scratch_types=tuple(scratch),
        compiler_params=pltpu.CompilerParams(use_tc_tiling_on_sc=False),
    )


_agg_deg = _make_agg(True)
_agg = _make_agg(False)

BR = 2000


def _tc1_body(x_ref, p_ref, dg_ref, w_ref, b_ref, h_ref):
    deg = jnp.maximum(dg_ref[0, :, :1] + dg_ref[1, :, :1], 1.0)
    agg = jnp.concatenate([p_ref[0], p_ref[1]], axis=1) / deg
    t = x_ref[...] + agg
    res = jnp.maximum(
        jnp.dot(t, w_ref[...], preferred_element_type=jnp.float32) + b_ref[...], 0.0
    )
    h_ref[0] = res[:, :DH]
    h_ref[1] = res[:, DH:]


_tc1 = pl.pallas_call(
    _tc1_body,
    grid=(N // BR,),
    in_specs=[
        pl.BlockSpec((BR, D), lambda i: (i, 0)),
        pl.BlockSpec((NC, BR, DH), lambda i: (0, i, 0)),
        pl.BlockSpec((NC, BR, DW), lambda i: (0, i, 0)),
        pl.BlockSpec((D, H), lambda i: (0, 0)),
        pl.BlockSpec((1, H), lambda i: (0, 0)),
    ],
    out_specs=pl.BlockSpec((NC, BR, DH), lambda i: (0, i, 0)),
    out_shape=jax.ShapeDtypeStruct((NC, N, DH), jnp.float32),
)


def _tc2_body(h_ref, p_ref, dg_ref, w_ref, b_ref, wc_ref, bc_ref, o_ref, mx_ref):
    i = pl.program_id(0)
    deg = jnp.maximum(dg_ref[0, :, :1] + dg_ref[1, :, :1], 1.0)
    h = jnp.concatenate([h_ref[0], h_ref[1]], axis=1)
    t = h + jnp.concatenate([p_ref[0], p_ref[1]], axis=1) / deg
    h2 = jnp.dot(t, w_ref[...], preferred_element_type=jnp.float32) + b_ref[...]
    bm = jnp.max(h2, axis=0, keepdims=True)

    @pl.when(i == 0)
    def _():
        mx_ref[...] = bm

    @pl.when(i > 0)
    def _():
        mx_ref[...] = jnp.maximum(mx_ref[...], bm)

    @pl.when(i == pl.num_programs(0) - 1)
    def _():
        o_ref[...] = (
            jnp.dot(mx_ref[...], wc_ref[...], preferred_element_type=jnp.float32)
            + bc_ref[...]
        )


_tc2 = pl.pallas_call(
    _tc2_body,
    grid=(N // BR,),
    in_specs=[
        pl.BlockSpec((NC, BR, DH), lambda i: (0, i, 0)),
        pl.BlockSpec((NC, BR, DH), lambda i: (0, i, 0)),
        pl.BlockSpec((NC, BR, DW), lambda i: (0, i, 0)),
        pl.BlockSpec((H, H), lambda i: (0, 0)),
        pl.BlockSpec((1, H), lambda i: (0, 0)),
        pl.BlockSpec((H, C), lambda i: (0, 0)),
        pl.BlockSpec((1, C), lambda i: (0, 0)),
    ],
    out_specs=pl.BlockSpec((1, C), lambda i: (0, 0)),
    out_shape=jax.ShapeDtypeStruct((1, C), jnp.float32),
    scratch_shapes=[pltpu.VMEM((1, H), jnp.float32)],
)


@jax.jit
def kernel(x, edge_index, W1, b1, W2, b2, Wc, bc):
    src = edge_index[0]
    dst = edge_index[1]
    pad = E_PAD - E
    srcp = jnp.concatenate([src, jnp.zeros((pad,), jnp.int32)]).reshape(NS, NCH, K)
    dstp = jnp.concatenate([dst, jnp.full((pad,), N, jnp.int32)]).reshape(NS, NCH, K)
    z2d = jnp.zeros((ZR, DH), jnp.float32)
    zdeg = jnp.zeros((ZR, DW), jnp.float32)
    ones = jnp.ones((K, DW), jnp.float32)

    xt = x.reshape(N, NC, DH).transpose(1, 0, 2)
    p1, degp = _agg_deg(xt, srcp, dstp, z2d, zdeg, ones)
    h1 = _tc1(x, p1, degp, W1, b1.reshape(1, H))
    p2 = _agg(h1, srcp, dstp, z2d, zdeg, ones)
    return _tc2(h1, p2, degp, W2, b2.reshape(1, H), Wc, bc.reshape(1, C))


# speedup vs baseline: 1.0019x; 1.0019x over previous
import functools

import jax
import jax.numpy as jnp
from jax import lax
from jax.experimental import pallas as pl
from jax.experimental.pallas import tpu as pltpu
from jax.experimental.pallas import tpu_sc as plsc

N = 10000
D = 128
H = 128
C = 16
E = 320000

NC = 2
NS = 16
NW = NC * NS

N_PAD = 10112
ZR = N_PAD // NS
K = 128
NCH = 160
EPT = NCH * K
E_PAD = NS * EPT
DH = D // 2
KB = 5

_mesh = plsc.VectorSubcoreMesh(core_axis_name="c", subcore_axis_name="s")


DW = 8


def _make_agg(with_deg):
    out_type = [jax.ShapeDtypeStruct((NC, N_PAD, DH), jnp.float32)]
    scratch = [
        pltpu.VMEM((KB, K), jnp.int32),
        pltpu.VMEM((NCH, K), jnp.int32),
    ] + [pltpu.VMEM((K, DH), jnp.float32)] * KB + [
        pltpu.VMEM_SHARED((N_PAD, DH), jnp.float32),
    ] + [pltpu.SemaphoreType.DMA] * (3 * KB)
    if with_deg:
        out_type.append(jax.ShapeDtypeStruct((NC, N_PAD, DW), jnp.float32))
        scratch.append(pltpu.VMEM((K, DW), jnp.float32))
        scratch.append(pltpu.VMEM_SHARED((N_PAD, DW), jnp.float32))

    def body(h_hbm, srcr, dstr, z2d, zdeg, ones_hbm, *rest):
        if with_deg:
            out_hbm, deg_hbm, sidxr, dall = rest[:4]
            rows = rest[4:4 + KB]
            acc = rest[4 + KB]
            sems = rest[5 + KB:5 + 4 * KB]
            onesb, dacc = rest[5 + 4 * KB:]
        else:
            out_hbm, sidxr, dall = rest[:3]
            rows = rest[3:3 + KB]
            acc = rest[3 + KB]
            sems = rest[4 + KB:4 + 4 * KB]
        isem = sems[:KB]
        gsem = sems[KB:2 * KB]
        ssem = sems[2 * KB:]
        c = lax.axis_index("c")
        s = lax.axis_index("s")

        pltpu.sync_copy(dstr.at[s], dall)
        pltpu.sync_copy(z2d, acc.at[pl.ds(s * ZR, ZR)])
        if with_deg:
            pltpu.sync_copy(ones_hbm, onesb)
            pltpu.sync_copy(zdeg, dacc.at[pl.ds(s * ZR, ZR)])
        plsc.subcore_barrier()

        tbl = h_hbm.at[c]
        gdon = tbl.at[pl.ds(0, K)]
        idon = srcr.at[s, 0]

        def sdrain(b):
            pltpu.make_async_copy(rows[b], acc.at[dall.at[0]], ssem[b]).wait()

        for b in range(KB):
            pltpu.async_copy(srcr.at[s, b], sidxr.at[b], isem[b])
        for b in range(2):
            pltpu.make_async_copy(idon, sidxr.at[b], isem[b]).wait()
            pltpu.async_copy(tbl.at[sidxr.at[b]], rows[b], gsem[b])

        def it(i, carry):
            for b in range(KB):
                j = i * KB + b
                bg = (b + 2) % KB
                bs = (b + KB - 2) % KB
                pltpu.make_async_copy(gdon, rows[b], gsem[b]).wait()

                @pl.when(j + KB < NCH)
                def _():
                    pltpu.async_copy(srcr.at[s, j + KB], sidxr.at[b], isem[b])

                pltpu.async_copy(rows[b], acc.at[dall.at[j]], ssem[b], add=True)
                if with_deg:

                    @pl.when(c == (i + b) % 2)
                    def _():
                        pltpu.sync_copy(onesb, dacc.at[dall.at[j]], add=True)

                @pl.when(j + 2 < NCH)
                def _():
                    @pl.when(j >= 2)
                    def _():
                        sdrain(bs)

                    pltpu.make_async_copy(idon, sidxr.at[bg], isem[bg]).wait()
                    pltpu.async_copy(tbl.at[sidxr.at[bg]], rows[bg], gsem[bg])

            return carry

        lax.fori_loop(0, NCH // KB, it, 0)
        for t in range(NCH - 4, NCH):
            sdrain(t % KB)
        plsc.subcore_barrier()
        pltpu.sync_copy(acc.at[pl.ds(s * ZR, ZR)], out_hbm.at[c, pl.ds(s * ZR, ZR)])
        if with_deg:
            pltpu.sync_copy(dacc.at[pl.ds(s * ZR, ZR)], deg_hbm.at[c, pl.ds(s * ZR, ZR)])

    return pl.kernel(
        body,
        out_type=out_type if with_deg else out_type[0],
        mesh=_mesh,
        scratch_types=tuple(scratch),
        compiler_params=pltpu.CompilerParams(use_tc_tiling_on_sc=False),
    )


_agg_deg = _make_agg(True)
_agg = _make_agg(False)

BR = 2000


def _tc1_body(x_ref, p_ref, dg_ref, w_ref, b_ref, h_ref):
    deg = jnp.maximum(dg_ref[0, :, :1] + dg_ref[1, :, :1], 1.0)
    agg = jnp.concatenate([p_ref[0], p_ref[1]], axis=1) / deg
    t = x_ref[...] + agg
    res = jnp.maximum(
        jnp.dot(t, w_ref[...], preferred_element_type=jnp.float32) + b_ref[...], 0.0
    )
    h_ref[0] = res[:, :DH]
    h_ref[1] = res[:, DH:]


_tc1 = pl.pallas_call(
    _tc1_body,
    grid=(N // BR,),
    in_specs=[
        pl.BlockSpec((BR, D), lambda i: (i, 0)),
        pl.BlockSpec((NC, BR, DH), lambda i: (0, i, 0)),
        pl.BlockSpec((NC, BR, DW), lambda i: (0, i, 0)),
        pl.BlockSpec((D, H), lambda i: (0, 0)),
        pl.BlockSpec((1, H), lambda i: (0, 0)),
    ],
    out_specs=pl.BlockSpec((NC, BR, DH), lambda i: (0, i, 0)),
    out_shape=jax.ShapeDtypeStruct((NC, N, DH), jnp.float32),
)


def _tc2_body(h_ref, p_ref, dg_ref, w_ref, b_ref, wc_ref, bc_ref, o_ref, mx_ref):
    i = pl.program_id(0)
    deg = jnp.maximum(dg_ref[0, :, :1] + dg_ref[1, :, :1], 1.0)
    h = jnp.concatenate([h_ref[0], h_ref[1]], axis=1)
    t = h + jnp.concatenate([p_ref[0], p_ref[1]], axis=1) / deg
    h2 = jnp.dot(t, w_ref[...], preferred_element_type=jnp.float32) + b_ref[...]
    bm = jnp.max(h2, axis=0, keepdims=True)

    @pl.when(i == 0)
    def _():
        mx_ref[...] = bm

    @pl.when(i > 0)
    def _():
        mx_ref[...] = jnp.maximum(mx_ref[...], bm)

    @pl.when(i == pl.num_programs(0) - 1)
    def _():
        o_ref[...] = (
            jnp.dot(mx_ref[...], wc_ref[...], preferred_element_type=jnp.float32)
            + bc_ref[...]
        )


_tc2 = pl.pallas_call(
    _tc2_body,
    grid=(N // BR,),
    in_specs=[
        pl.BlockSpec((NC, BR, DH), lambda i: (0, i, 0)),
        pl.BlockSpec((NC, BR, DH), lambda i: (0, i, 0)),
        pl.BlockSpec((NC, BR, DW), lambda i: (0, i, 0)),
        pl.BlockSpec((H, H), lambda i: (0, 0)),
        pl.BlockSpec((1, H), lambda i: (0, 0)),
        pl.BlockSpec((H, C), lambda i: (0, 0)),
        pl.BlockSpec((1, C), lambda i: (0, 0)),
    ],
    out_specs=pl.BlockSpec((1, C), lambda i: (0, 0)),
    out_shape=jax.ShapeDtypeStruct((1, C), jnp.float32),
    scratch_shapes=[pltpu.VMEM((1, H), jnp.float32)],
)


@jax.jit
def kernel(x, edge_index, W1, b1, W2, b2, Wc, bc):
    src = edge_index[0]
    dst = edge_index[1]
    pad = E_PAD - E
    srcp = jnp.concatenate([src, jnp.zeros((pad,), jnp.int32)]).reshape(NS, NCH, K)
    dstp = jnp.concatenate([dst, jnp.full((pad,), N, jnp.int32)]).reshape(NS, NCH, K)
    z2d = jnp.zeros((ZR, DH), jnp.float32)
    zdeg = jnp.zeros((ZR, DW), jnp.float32)
    ones = jnp.ones((K, DW), jnp.float32)

    xt = x.reshape(N, NC, DH).transpose(1, 0, 2)
    p1, degp = _agg_deg(xt, srcp, dstp, z2d, zdeg, ones)
    h1 = _tc1(x, p1, degp, W1, b1.reshape(1, H))
    p2 = _agg(h1, srcp, dstp, z2d, zdeg, ones)
    return _tc2(h1, p2, degp, W2, b2.reshape(1, H), Wc, bc.reshape(1, C))


# speedup vs baseline: 1.0062x; 1.0043x over previous
import functools

import jax
import jax.numpy as jnp
from jax import lax
from jax.experimental import pallas as pl
from jax.experimental.pallas import tpu as pltpu
from jax.experimental.pallas import tpu_sc as plsc

N = 10000
D = 128
H = 128
C = 16
E = 320000

NC = 2
NS = 16
NW = NC * NS

N_PAD = 10112
ZR = N_PAD // NS
K = 128
NCH = 160
EPT = NCH * K
E_PAD = NS * EPT
DH = D // 2
KB = 5

_mesh = plsc.VectorSubcoreMesh(core_axis_name="c", subcore_axis_name="s")


DW = 8


def _make_agg(with_deg):
    out_type = [jax.ShapeDtypeStruct((NC, N_PAD, DH), jnp.float32)]
    scratch = [
        pltpu.VMEM((KB, K), jnp.int32),
        pltpu.VMEM((NCH, K), jnp.int32),
    ] + [pltpu.VMEM((K, DH), jnp.float32)] * KB + [
        pltpu.VMEM_SHARED((N_PAD, DH), jnp.float32),
    ] + [pltpu.SemaphoreType.DMA] * (3 * KB)
    if with_deg:
        out_type.append(jax.ShapeDtypeStruct((NC, N_PAD, DW), jnp.float32))
        scratch.append(pltpu.VMEM((K, DW), jnp.float32))
        scratch.append(pltpu.VMEM_SHARED((N_PAD, DW), jnp.float32))

    def body(h_hbm, srcr, dstr, z2d, zdeg, ones_hbm, *rest):
        if with_deg:
            out_hbm, deg_hbm, sidxr, dall = rest[:4]
            rows = rest[4:4 + KB]
            acc = rest[4 + KB]
            sems = rest[5 + KB:5 + 4 * KB]
            onesb, dacc = rest[5 + 4 * KB:]
        else:
            out_hbm, sidxr, dall = rest[:3]
            rows = rest[3:3 + KB]
            acc = rest[3 + KB]
            sems = rest[4 + KB:4 + 4 * KB]
        isem = sems[:KB]
        gsem = sems[KB:2 * KB]
        ssem = sems[2 * KB:]
        c = lax.axis_index("c")
        s = lax.axis_index("s")

        pltpu.sync_copy(dstr.at[s], dall)
        pltpu.sync_copy(z2d, acc.at[pl.ds(s * ZR, ZR)])
        if with_deg:
            pltpu.sync_copy(ones_hbm, onesb)
            pltpu.sync_copy(zdeg, dacc.at[pl.ds(s * ZR, ZR)])
        plsc.subcore_barrier()

        tbl = h_hbm.at[c]

        def grow(idx_ref):
            return tbl.at[idx_ref]

        gdon = tbl.at[pl.ds(0, K)]
        idon = srcr.at[s, 0]

        def sdrain(b):
            pltpu.make_async_copy(rows[b], acc.at[dall.at[0]], ssem[b]).wait()

        for b in range(KB):
            pltpu.async_copy(srcr.at[s, b], sidxr.at[b], isem[b])
        for b in range(2):
            pltpu.make_async_copy(idon, sidxr.at[b], isem[b]).wait()
            pltpu.async_copy(grow(sidxr.at[b]), rows[b], gsem[b])

        def it(i, carry):
            for b in range(KB):
                j = i * KB + b
                bg = (b + 2) % KB
                bs = (b + KB - 2) % KB
                pltpu.make_async_copy(gdon, rows[b], gsem[b]).wait()

                @pl.when(j + KB < NCH)
                def _():
                    pltpu.async_copy(srcr.at[s, j + KB], sidxr.at[b], isem[b])

                pltpu.async_copy(rows[b], acc.at[dall.at[j]], ssem[b], add=True)
                if with_deg:

                    @pl.when(c == (i + b) % 2)
                    def _():
                        pltpu.sync_copy(onesb, dacc.at[dall.at[j]], add=True)

                @pl.when(j + 2 < NCH)
                def _():
                    @pl.when(j >= 2)
                    def _():
                        sdrain(bs)

                    pltpu.make_async_copy(idon, sidxr.at[bg], isem[bg]).wait()
                    pltpu.async_copy(grow(sidxr.at[bg]), rows[bg], gsem[bg])

            return carry

        lax.fori_loop(0, NCH // KB, it, 0)
        for t in range(NCH - 4, NCH):
            sdrain(t % KB)
        plsc.subcore_barrier()
        pltpu.sync_copy(acc.at[pl.ds(s * ZR, ZR)], out_hbm.at[c, pl.ds(s * ZR, ZR)])
        if with_deg:
            pltpu.sync_copy(dacc.at[pl.ds(s * ZR, ZR)], deg_hbm.at[c, pl.ds(s * ZR, ZR)])

    return pl.kernel(
        body,
        out_type=out_type if with_deg else out_type[0],
        mesh=_mesh,
        scratch_types=tuple(scratch),
        compiler_params=pltpu.CompilerParams(use_tc_tiling_on_sc=False),
    )


_agg_deg = _make_agg(True)
_agg = _make_agg(False)

BR = 2000


def _tc1_body(x_ref, p_ref, dg_ref, w_ref, b_ref, h_ref):
    deg = jnp.maximum(dg_ref[0, :, :1] + dg_ref[1, :, :1], 1.0)
    agg = jnp.concatenate([p_ref[0], p_ref[1]], axis=1) / deg
    t = x_ref[...] + agg
    h_ref[...] = jnp.maximum(
        jnp.dot(t, w_ref[...], preferred_element_type=jnp.float32) + b_ref[...], 0.0
    )


_tc1 = pl.pallas_call(
    _tc1_body,
    grid=(N // BR,),
    in_specs=[
        pl.BlockSpec((BR, D), lambda i: (i, 0)),
        pl.BlockSpec((NC, BR, DH), lambda i: (0, i, 0)),
        pl.BlockSpec((NC, BR, DW), lambda i: (0, i, 0)),
        pl.BlockSpec((D, H), lambda i: (0, 0)),
        pl.BlockSpec((1, H), lambda i: (0, 0)),
    ],
    out_specs=pl.BlockSpec((BR, H), lambda i: (i, 0)),
    out_shape=jax.ShapeDtypeStruct((N, H), jnp.float32),
)


def _tc2_body(h_ref, p_ref, dg_ref, w_ref, b_ref, wc_ref, bc_ref, o_ref, mx_ref):
    i = pl.program_id(0)
    deg = jnp.maximum(dg_ref[0, :, :1] + dg_ref[1, :, :1], 1.0)
    t = h_ref[...] + jnp.concatenate([p_ref[0], p_ref[1]], axis=1) / deg
    h2 = jnp.dot(t, w_ref[...], preferred_element_type=jnp.float32) + b_ref[...]
    bm = jnp.max(h2, axis=0, keepdims=True)

    @pl.when(i == 0)
    def _():
        mx_ref[...] = bm

    @pl.when(i > 0)
    def _():
        mx_ref[...] = jnp.maximum(mx_ref[...], bm)

    @pl.when(i == pl.num_programs(0) - 1)
    def _():
        o_ref[...] = (
            jnp.dot(mx_ref[...], wc_ref[...], preferred_element_type=jnp.float32)
            + bc_ref[...]
        )


_tc2 = pl.pallas_call(
    _tc2_body,
    grid=(N // BR,),
    in_specs=[
        pl.BlockSpec((BR, H), lambda i: (i, 0)),
        pl.BlockSpec((NC, BR, DH), lambda i: (0, i, 0)),
        pl.BlockSpec((NC, BR, DW), lambda i: (0, i, 0)),
        pl.BlockSpec((H, H), lambda i: (0, 0)),
        pl.BlockSpec((1, H), lambda i: (0, 0)),
        pl.BlockSpec((H, C), lambda i: (0, 0)),
        pl.BlockSpec((1, C), lambda i: (0, 0)),
    ],
    out_specs=pl.BlockSpec((1, C), lambda i: (0, 0)),
    out_shape=jax.ShapeDtypeStruct((1, C), jnp.float32),
    scratch_shapes=[pltpu.VMEM((1, H), jnp.float32)],
)


@jax.jit
def kernel(x, edge_index, W1, b1, W2, b2, Wc, bc):
    src = edge_index[0]
    dst = edge_index[1]
    pad = E_PAD - E
    srcp = jnp.concatenate([src, jnp.zeros((pad,), jnp.int32)]).reshape(NS, NCH, K)
    dstp = jnp.concatenate([dst, jnp.full((pad,), N, jnp.int32)]).reshape(NS, NCH, K)
    z2d = jnp.zeros((ZR, DH), jnp.float32)
    zdeg = jnp.zeros((ZR, DW), jnp.float32)
    ones = jnp.ones((K, DW), jnp.float32)

    xt = x.reshape(N, NC, DH).transpose(1, 0, 2)
    p1, degp = _agg_deg(xt, srcp, dstp, z2d, zdeg, ones)
    h1 = _tc1(x, p1, degp, W1, b1.reshape(1, H))
    h1t = h1.reshape(N, NC, DH).transpose(1, 0, 2)
    p2 = _agg(h1t, srcp, dstp, z2d, zdeg, ones)
    return _tc2(h1, p2, degp, W2, b2.reshape(1, H), Wc, bc.reshape(1, C))
